# baseline (speedup 1.0000x reference)
import jax
import jax.numpy as jnp
from jax import lax
from jax.experimental import pallas as pl

B = 4
S = 2048
F = 1024
NQ = 32


def _tc_quarter_body(x_ref, m8r, v8r, m16r, v16r, m32r, v32r):
    x = x_ref[...]
    q = pl.program_id(0)
    is_last = (q % 8) == 7
    x2 = x * x

    cs1 = [jnp.sum(x[c * 64:(c + 1) * 64], axis=0, keepdims=True)
           for c in range(4)]
    cs2 = [jnp.sum(x2[c * 64:(c + 1) * 64], axis=0, keepdims=True)
           for c in range(4)]
    corr = jnp.where(is_last, 0.01, 0.0)
    v = x[255:256]
    cs1[3] = cs1[3] - corr * v
    cs2[3] = cs2[3] - corr * (v * v)

    def stats(s1, s2, wt):
        mean = s1 / wt
        var = jnp.sqrt(jnp.maximum(s2 / wt - mean * mean, 0.0))
        return mean, var

    w32l = jnp.where(is_last, 63.99, 64.0)
    w16l = jnp.where(is_last, 127.99, 128.0)
    w8l = jnp.where(is_last, 255.99, 256.0)

    m32s, v32s = [], []
    for c in range(4):
        m, v = stats(cs1[c], cs2[c], w32l if c == 3 else 64.0)
        m32s.append(m)
        v32s.append(v)
    m32r[0] = jnp.concatenate(m32s, axis=0)
    v32r[0] = jnp.concatenate(v32s, axis=0)

    p1 = [cs1[0] + cs1[1], cs1[2] + cs1[3]]
    p2 = [cs2[0] + cs2[1], cs2[2] + cs2[3]]
    m16s, v16s = [], []
    for i in range(2):
        m, v = stats(p1[i], p2[i], w16l if i == 1 else 128.0)
        m16s.append(m)
        v16s.append(v)
    m16r[0] = jnp.concatenate(m16s, axis=0)
    v16r[0] = jnp.concatenate(v16s, axis=0)

    m8, v8 = stats(p1[0] + p1[1], p2[0] + p2[1], w8l)
    m8r[0] = m8
    v8r[0] = v8


@jax.jit
def kernel(x, blocks_score_0, blocks_score_1, blocks_score_2):
    del blocks_score_0, blocks_score_1, blocks_score_2
    f32 = jnp.float32
    m8, v8, m16, v16, m32, v32 = pl.pallas_call(
        _tc_quarter_body,
        grid=(NQ,),
        in_specs=[pl.BlockSpec((256, F), lambda q: (q, 0))],
        out_specs=[
            pl.BlockSpec((1, 1, F), lambda q: (q, 0, 0)),
            pl.BlockSpec((1, 1, F), lambda q: (q, 0, 0)),
            pl.BlockSpec((1, 2, F), lambda q: (q, 0, 0)),
            pl.BlockSpec((1, 2, F), lambda q: (q, 0, 0)),
            pl.BlockSpec((1, 4, F), lambda q: (q, 0, 0)),
            pl.BlockSpec((1, 4, F), lambda q: (q, 0, 0)),
        ],
        out_shape=[
            jax.ShapeDtypeStruct((NQ, 1, F), f32),
            jax.ShapeDtypeStruct((NQ, 1, F), f32),
            jax.ShapeDtypeStruct((NQ, 2, F), f32),
            jax.ShapeDtypeStruct((NQ, 2, F), f32),
            jax.ShapeDtypeStruct((NQ, 4, F), f32),
            jax.ShapeDtypeStruct((NQ, 4, F), f32),
        ],
    )(x.reshape(B * S, F))
    return jnp.concatenate(
        [m8.reshape(B, 8, F), v8.reshape(B, 8, F),
         m16.reshape(B, 16, F), v16.reshape(B, 16, F),
         m32.reshape(B, 32, F), v32.reshape(B, 32, F)], axis=1)


# speedup vs baseline: 1.4840x; 1.4840x over previous
import jax
import jax.numpy as jnp
from jax import lax
from jax.experimental import pallas as pl

B = 4
S = 2048
F = 1024
QPB = 4
RPB = 256 * QPB
NSTEP = (B * S) // RPB


def _tc_body(x_ref, m8r, v8r, m16r, v16r, m32r, v32r):
    x = x_ref[...]
    x2 = x * x
    step = pl.program_id(0)

    m8s, v8s, m16s, v16s, m32s, v32s = [], [], [], [], [], []
    for c4 in range(QPB):
        qg = step * QPB + c4
        is_last = (qg % 8) == 7
        base = c4 * 256
        cs1 = [jnp.sum(x[base + c * 64:base + (c + 1) * 64], axis=0,
                       keepdims=True) for c in range(4)]
        cs2 = [jnp.sum(x2[base + c * 64:base + (c + 1) * 64], axis=0,
                       keepdims=True) for c in range(4)]
        corr = jnp.where(is_last, 0.01, 0.0)
        v = x[base + 255:base + 256]
        cs1[3] = cs1[3] - corr * v
        cs2[3] = cs2[3] - corr * (v * v)

        def stats(s1, s2, wt):
            mean = s1 / wt
            var = jnp.sqrt(jnp.maximum(s2 / wt - mean * mean, 0.0))
            return mean, var

        w32l = jnp.where(is_last, 63.99, 64.0)
        w16l = jnp.where(is_last, 127.99, 128.0)
        w8l = jnp.where(is_last, 255.99, 256.0)

        for c in range(4):
            m, v_ = stats(cs1[c], cs2[c], w32l if c == 3 else 64.0)
            m32s.append(m)
            v32s.append(v_)
        p1 = [cs1[0] + cs1[1], cs1[2] + cs1[3]]
        p2 = [cs2[0] + cs2[1], cs2[2] + cs2[3]]
        for i in range(2):
            m, v_ = stats(p1[i], p2[i], w16l if i == 1 else 128.0)
            m16s.append(m)
            v16s.append(v_)
        m, v_ = stats(p1[0] + p1[1], p2[0] + p2[1], w8l)
        m8s.append(m)
        v8s.append(v_)

    m8r[0] = jnp.concatenate(m8s, axis=0)
    v8r[0] = jnp.concatenate(v8s, axis=0)
    m16r[0] = jnp.concatenate(m16s, axis=0)
    v16r[0] = jnp.concatenate(v16s, axis=0)
    m32r[0] = jnp.concatenate(m32s, axis=0)
    v32r[0] = jnp.concatenate(v32s, axis=0)


@jax.jit
def kernel(x, blocks_score_0, blocks_score_1, blocks_score_2):
    del blocks_score_0, blocks_score_1, blocks_score_2
    f32 = jnp.float32
    m8, v8, m16, v16, m32, v32 = pl.pallas_call(
        _tc_body,
        grid=(NSTEP,),
        in_specs=[pl.BlockSpec((RPB, F), lambda s: (s, 0))],
        out_specs=[
            pl.BlockSpec((1, QPB, F), lambda s: (s, 0, 0)),
            pl.BlockSpec((1, QPB, F), lambda s: (s, 0, 0)),
            pl.BlockSpec((1, 2 * QPB, F), lambda s: (s, 0, 0)),
            pl.BlockSpec((1, 2 * QPB, F), lambda s: (s, 0, 0)),
            pl.BlockSpec((1, 4 * QPB, F), lambda s: (s, 0, 0)),
            pl.BlockSpec((1, 4 * QPB, F), lambda s: (s, 0, 0)),
        ],
        out_shape=[
            jax.ShapeDtypeStruct((NSTEP, QPB, F), f32),
            jax.ShapeDtypeStruct((NSTEP, QPB, F), f32),
            jax.ShapeDtypeStruct((NSTEP, 2 * QPB, F), f32),
            jax.ShapeDtypeStruct((NSTEP, 2 * QPB, F), f32),
            jax.ShapeDtypeStruct((NSTEP, 4 * QPB, F), f32),
            jax.ShapeDtypeStruct((NSTEP, 4 * QPB, F), f32),
        ],
    )(x.reshape(B * S, F))
    return jnp.concatenate(
        [m8.reshape(B, 8, F), v8.reshape(B, 8, F),
         m16.reshape(B, 16, F), v16.reshape(B, 16, F),
         m32.reshape(B, 32, F), v32.reshape(B, 32, F)], axis=1)
